# Initial kernel scaffold; baseline (speedup 1.0000x reference)
import jax
import jax.numpy as jnp
from jax.experimental import pallas as pl


def kernel(img, mask, weight, gamma, beta):
    raise NotImplementedError("write your pallas kernel here")


# speedup vs baseline: 5.2386x; 5.2386x over previous
import functools

import jax
import jax.numpy as jnp
from jax import lax
from jax.experimental import pallas as pl
from jax.experimental.pallas import tpu as pltpu


def _conv_stats_kernel(img_ref, mask_ref, w_ref, out_ref, nmask_ref, stats_ref,
                       *, cout, H, W):
    L = H * W
    PAD = W + 2
    x = img_ref[0]
    m = mask_ref[0]
    xm = (x * m).astype(jnp.bfloat16)

    xp = jnp.pad(xm, ((0, 0), (PAD, PAD)))
    lane = lax.broadcasted_iota(jnp.int32, (1, L + 2 * PAD), 1)
    wcol = (lane - PAD) % W
    keepL = wcol != (W - 1)
    keepR = wcol != 0
    zb = jnp.zeros((), jnp.bfloat16)
    xpL = jnp.where(keepL, xp, zb)
    xpR = jnp.where(keepR, xp, zb)
    mp = jnp.pad(m, ((0, 0), (PAD, PAD)))
    mpL = jnp.where(keepL, mp, 0.0)
    mpR = jnp.where(keepR, mp, 0.0)

    acc = jnp.zeros((cout, L), jnp.float32)
    ws = jnp.zeros((1, L), jnp.float32)
    t = 0
    for di in range(3):
        for dj in range(3):
            s = PAD + (di - 1) * W + (dj - 1)
            src = xpL if dj == 0 else (xpR if dj == 2 else xp)
            msrc = mpL if dj == 0 else (mpR if dj == 2 else mp)
            acc = acc + jnp.dot(w_ref[t], src[:, s:s + L],
                                preferred_element_type=jnp.float32)
            ws = ws + msrc[:, s:s + L]
            t += 1

    hole = ws == 0.0
    inv = jnp.where(hole, 0.0, 9.0 / jnp.where(hole, 1.0, ws))
    out = acc * inv

    out_ref[...] = out.astype(jnp.bfloat16)[None]
    nmask_ref[...] = jnp.where(hole, 0.0, 1.0)[None]

    s1 = jnp.sum(out, axis=1, keepdims=True)
    s2 = jnp.sum(out * out, axis=1, keepdims=True)
    stats_ref[...] = jnp.concatenate([s1, s2], axis=1)[None]


def _bn_relu_kernel(out_ref, nm_ref, sc_ref, sh_ref, y_ref, nmo_ref, *, cout):
    x = out_ref[0].astype(jnp.float32)
    y_ref[...] = jnp.maximum(x * sc_ref[...] + sh_ref[...], 0.0)[None]
    nm = nm_ref[0]
    nmo_ref[...] = jnp.broadcast_to(nm, (cout, nm.shape[1]))[None]


def kernel(img, mask, weight, gamma, beta):
    N, Cin, H, W = img.shape
    Cout = weight.shape[0]
    L = H * W
    P = N * L

    img2 = img.reshape(N, Cin, L)
    mask2 = mask[:, :1].reshape(N, 1, L)
    w_taps = weight.reshape(Cout, Cin, 9).transpose(2, 0, 1).astype(jnp.bfloat16)

    cparams = pltpu.CompilerParams(
        dimension_semantics=("parallel",),
        vmem_limit_bytes=100 * 1024 * 1024)

    kern1 = functools.partial(_conv_stats_kernel, cout=Cout, H=H, W=W)
    out_t, nmask_t, stats = pl.pallas_call(
        kern1,
        out_shape=(jax.ShapeDtypeStruct((N, Cout, L), jnp.bfloat16),
                   jax.ShapeDtypeStruct((N, 1, L), jnp.float32),
                   jax.ShapeDtypeStruct((N, Cout, 2), jnp.float32)),
        grid=(N,),
        in_specs=[pl.BlockSpec((1, Cin, L), lambda i: (i, 0, 0)),
                  pl.BlockSpec((1, 1, L), lambda i: (i, 0, 0)),
                  pl.BlockSpec((9, Cout, Cin), lambda i: (0, 0, 0))],
        out_specs=(pl.BlockSpec((1, Cout, L), lambda i: (i, 0, 0)),
                   pl.BlockSpec((1, 1, L), lambda i: (i, 0, 0)),
                   pl.BlockSpec((1, Cout, 2), lambda i: (i, 0, 0))),
        compiler_params=cparams,
        cost_estimate=pl.CostEstimate(
            flops=2 * P * Cin * 9 * Cout,
            transcendentals=P,
            bytes_accessed=4 * P * Cin + 4 * P // 64 + 2 * P * Cout),
    )(img2, mask2, w_taps)

    s1 = jnp.sum(stats[:, :, 0], axis=0)
    s2 = jnp.sum(stats[:, :, 1], axis=0)
    mean = s1 / P
    var = jnp.maximum(s2 / P - mean * mean, 0.0)
    inv_std = lax.rsqrt(var + 1e-5)
    sc = (gamma * inv_std).reshape(Cout, 1).astype(jnp.float32)
    sh = (beta - mean * gamma * inv_std).reshape(Cout, 1).astype(jnp.float32)

    y, nmo = pl.pallas_call(
        functools.partial(_bn_relu_kernel, cout=Cout),
        out_shape=(jax.ShapeDtypeStruct((N, Cout, L), jnp.float32),
                   jax.ShapeDtypeStruct((N, Cout, L), jnp.float32)),
        grid=(N,),
        in_specs=[pl.BlockSpec((1, Cout, L), lambda i: (i, 0, 0)),
                  pl.BlockSpec((1, 1, L), lambda i: (i, 0, 0)),
                  pl.BlockSpec((Cout, 1), lambda i: (0, 0)),
                  pl.BlockSpec((Cout, 1), lambda i: (0, 0))],
        out_specs=(pl.BlockSpec((1, Cout, L), lambda i: (i, 0, 0)),
                   pl.BlockSpec((1, Cout, L), lambda i: (i, 0, 0))),
        compiler_params=cparams,
    )(out_t, nmask_t, sc, sh)

    out = y.reshape(N, Cout, H, W)
    new_mask = nmo.reshape(N, Cout, H, W)
    return out, new_mask


# speedup vs baseline: 10.5499x; 2.0139x over previous
import functools

import jax
import jax.numpy as jnp
from jax import lax
from jax.experimental import pallas as pl
from jax.experimental.pallas import tpu as pltpu


def _conv_stats_kernel(img_ref, mask_ref, w_ref, out_ref, nmask_ref, stats_ref,
                       *, cout, H, W):
    L = H * W
    PAD = W + 2
    x4 = img_ref[0]
    m2 = mask_ref[0, 0]
    xm = (x4 * m2[None]).astype(jnp.bfloat16)
    xm = xm.reshape(x4.shape[0], L)
    m = m2.reshape(1, L)

    xp = jnp.pad(xm, ((0, 0), (PAD, PAD)))
    lane = lax.broadcasted_iota(jnp.int32, (1, L + 2 * PAD), 1)
    wcol = (lane - PAD) % W
    keepL = wcol != (W - 1)
    keepR = wcol != 0
    zb = jnp.zeros((), jnp.bfloat16)
    xpL = jnp.where(keepL, xp, zb)
    xpR = jnp.where(keepR, xp, zb)
    mp = jnp.pad(m, ((0, 0), (PAD, PAD)))
    mpL = jnp.where(keepL, mp, 0.0)
    mpR = jnp.where(keepR, mp, 0.0)

    acc = jnp.zeros((cout, L), jnp.float32)
    ws = jnp.zeros((1, L), jnp.float32)
    t = 0
    for di in range(3):
        for dj in range(3):
            s = PAD + (di - 1) * W + (dj - 1)
            src = xpL if dj == 0 else (xpR if dj == 2 else xp)
            msrc = mpL if dj == 0 else (mpR if dj == 2 else mp)
            acc = acc + jnp.dot(w_ref[t], src[:, s:s + L],
                                preferred_element_type=jnp.float32)
            ws = ws + msrc[:, s:s + L]
            t += 1

    hole = ws == 0.0
    inv = jnp.where(hole, 0.0, 9.0 / jnp.where(hole, 1.0, ws))
    out = acc * inv

    s1 = jnp.sum(out, axis=1, keepdims=True)
    s2 = jnp.sum(out * out, axis=1, keepdims=True)
    stats_ref[...] = jnp.concatenate([s1, s2], axis=1)[None]

    out_ref[...] = out.astype(jnp.bfloat16).reshape(cout, H, W)[None]
    nm = jnp.where(hole, 0.0, 1.0)
    nmask_ref[...] = nm.reshape(H, W)[None, None]


def _bn_relu_kernel(out_ref, nm_ref, sc_ref, sh_ref, y_ref, nmo_ref, *, cout):
    x = out_ref[0].astype(jnp.float32)
    sc = sc_ref[0][:, :, None]
    sh = sh_ref[0][:, :, None]
    y_ref[...] = jnp.maximum(x * sc + sh, 0.0)[None]
    nm = nm_ref[0]
    nmo_ref[...] = jnp.broadcast_to(nm, x.shape)[None]


def kernel(img, mask, weight, gamma, beta):
    N, Cin, H, W = img.shape
    Cout = weight.shape[0]
    L = H * W
    P = N * L

    mask1 = mask[:, :1]
    w_taps = weight.reshape(Cout, Cin, 9).transpose(2, 0, 1).astype(jnp.bfloat16)

    cparams = pltpu.CompilerParams(
        dimension_semantics=("parallel",),
        vmem_limit_bytes=100 * 1024 * 1024)

    kern1 = functools.partial(_conv_stats_kernel, cout=Cout, H=H, W=W)
    out_t, nmask_t, stats = pl.pallas_call(
        kern1,
        out_shape=(jax.ShapeDtypeStruct((N, Cout, H, W), jnp.bfloat16),
                   jax.ShapeDtypeStruct((N, 1, H, W), jnp.float32),
                   jax.ShapeDtypeStruct((N, Cout, 2), jnp.float32)),
        grid=(N,),
        in_specs=[pl.BlockSpec((1, Cin, H, W), lambda i: (i, 0, 0, 0)),
                  pl.BlockSpec((1, 1, H, W), lambda i: (i, 0, 0, 0)),
                  pl.BlockSpec((9, Cout, Cin), lambda i: (0, 0, 0))],
        out_specs=(pl.BlockSpec((1, Cout, H, W), lambda i: (i, 0, 0, 0)),
                   pl.BlockSpec((1, 1, H, W), lambda i: (i, 0, 0, 0)),
                   pl.BlockSpec((1, Cout, 2), lambda i: (i, 0, 0))),
        compiler_params=cparams,
        cost_estimate=pl.CostEstimate(
            flops=2 * P * Cin * 9 * Cout,
            transcendentals=P,
            bytes_accessed=4 * P * Cin + 4 * P // 64 + 2 * P * Cout),
    )(img, mask1, w_taps)

    s1 = jnp.sum(stats[:, :, 0], axis=0)
    s2 = jnp.sum(stats[:, :, 1], axis=0)
    mean = s1 / P
    var = jnp.maximum(s2 / P - mean * mean, 0.0)
    inv_std = lax.rsqrt(var + 1e-5)
    sc = (gamma * inv_std).reshape(1, Cout, 1).astype(jnp.float32)
    sh = (beta - mean * gamma * inv_std).reshape(1, Cout, 1).astype(jnp.float32)

    y, nmo = pl.pallas_call(
        functools.partial(_bn_relu_kernel, cout=Cout),
        out_shape=(jax.ShapeDtypeStruct((N, Cout, H, W), jnp.float32),
                   jax.ShapeDtypeStruct((N, Cout, H, W), jnp.float32)),
        grid=(N,),
        in_specs=[pl.BlockSpec((1, Cout, H, W), lambda i: (i, 0, 0, 0)),
                  pl.BlockSpec((1, 1, H, W), lambda i: (i, 0, 0, 0)),
                  pl.BlockSpec((1, Cout, 1), lambda i: (0, 0, 0)),
                  pl.BlockSpec((1, Cout, 1), lambda i: (0, 0, 0))],
        out_specs=(pl.BlockSpec((1, Cout, H, W), lambda i: (i, 0, 0, 0)),
                   pl.BlockSpec((1, Cout, H, W), lambda i: (i, 0, 0, 0))),
        compiler_params=cparams,
    )(out_t, nmask_t, sc, sh)

    return y, nmo


# speedup vs baseline: 11.5062x; 1.0906x over previous
import functools

import jax
import jax.numpy as jnp
from jax import lax
from jax.experimental import pallas as pl
from jax.experimental.pallas import tpu as pltpu


def _conv_stats_kernel(img_ref, mask_ref, w_ref,
                       out_ref, nmask_ref, stats_ref, *, cout, H, W):
    L = H * W
    PAD = 2 * W
    x4 = img_ref[0]
    m2 = mask_ref[0, 0]
    xm = (x4 * m2[None]).astype(jnp.bfloat16)
    xm = xm.reshape(x4.shape[0], L)

    xp = jnp.pad(xm, ((0, 0), (PAD, PAD)))
    lane = lax.broadcasted_iota(jnp.int32, (1, L + 2 * PAD), 1)
    wcol = (lane - PAD) % W
    zb = jnp.zeros((), jnp.bfloat16)
    xpL = jnp.where(wcol != (W - 1), xp, zb)
    xpR = jnp.where(wcol != 0, xp, zb)

    srcs = (xpL, xp, xpR) * 3
    shifts = [PAD + (t // 3 - 1) * W + (t % 3 - 1) for t in range(9)]
    acc = jnp.dot(w_ref[4], xm,
                  preferred_element_type=jnp.float32)
    for t in (0, 1, 2, 3, 5, 6, 7, 8):
        acc = acc + jnp.dot(w_ref[t], srcs[t][:, shifts[t]:shifts[t] + L],
                            preferred_element_type=jnp.float32)

    m = m2.reshape(1, L)
    mp = jnp.pad(m, ((0, 0), (PAD, PAD)))
    mpL = jnp.where(wcol != (W - 1), mp, 0.0)
    mpR = jnp.where(wcol != 0, mp, 0.0)
    msrcs = (mpL, mp, mpR) * 3
    ws = msrcs[0][:, shifts[0]:shifts[0] + L]
    for t in range(1, 9):
        ws = ws + msrcs[t][:, shifts[t]:shifts[t] + L]

    hole = ws == 0.0
    inv = jnp.where(hole, 0.0, 9.0 / jnp.where(hole, 1.0, ws))
    out = acc * inv

    s1 = jnp.sum(out, axis=1, keepdims=True)
    s2 = jnp.sum(out * out, axis=1, keepdims=True)
    stats_ref[...] = jnp.concatenate([s1, s2], axis=1)[None]

    out_ref[...] = out.astype(jnp.bfloat16).reshape(cout, H, W)[None]
    nm = jnp.where(hole, 0.0, 1.0)
    nmask_ref[...] = nm.reshape(H, W)[None, None]


def _bn_relu_kernel(out_ref, nm_ref, stats_ref, gb_ref, y_ref, nmo_ref,
                    *, cout, P):
    s = jnp.sum(stats_ref[...], axis=0)
    mean = s[:, 0:1] / P
    var = jnp.maximum(s[:, 1:2] / P - mean * mean, 0.0)
    inv_std = lax.rsqrt(var + 1e-5)
    sc = gb_ref[:, 0:1] * inv_std
    sh = gb_ref[:, 1:2] - mean * sc
    x = out_ref[0].astype(jnp.float32)
    y_ref[...] = jnp.maximum(x * sc[:, :, None] + sh[:, :, None], 0.0)[None]
    nm = nm_ref[0]
    nmo_ref[...] = jnp.broadcast_to(nm, x.shape)[None]


def kernel(img, mask, weight, gamma, beta):
    N, Cin, H, W = img.shape
    Cout = weight.shape[0]
    L = H * W
    P = N * L

    w_taps = weight.reshape(Cout, Cin, 9).transpose(2, 0, 1).astype(jnp.bfloat16)
    gb = jnp.stack([gamma, beta], axis=1).astype(jnp.float32)

    cparams = pltpu.CompilerParams(
        dimension_semantics=("arbitrary",),
        vmem_limit_bytes=100 * 1024 * 1024)

    kern1 = functools.partial(_conv_stats_kernel, cout=Cout, H=H, W=W)
    out_t, nmask_t, stats = pl.pallas_call(
        kern1,
        out_shape=(jax.ShapeDtypeStruct((N, Cout, H, W), jnp.bfloat16),
                   jax.ShapeDtypeStruct((N, 1, H, W), jnp.float32),
                   jax.ShapeDtypeStruct((N, Cout, 2), jnp.float32)),
        grid=(N,),
        in_specs=[pl.BlockSpec((1, Cin, H, W), lambda i: (i, 0, 0, 0)),
                  pl.BlockSpec((1, 1, H, W), lambda i: (i, 0, 0, 0)),
                  pl.BlockSpec((9, Cout, Cin), lambda i: (0, 0, 0))],
        out_specs=(pl.BlockSpec((1, Cout, H, W), lambda i: (i, 0, 0, 0)),
                   pl.BlockSpec((1, 1, H, W), lambda i: (i, 0, 0, 0)),
                   pl.BlockSpec((1, Cout, 2), lambda i: (i, 0, 0))),
        compiler_params=cparams,
        cost_estimate=pl.CostEstimate(
            flops=2 * P * Cin * 9 * Cout,
            transcendentals=P,
            bytes_accessed=4 * P * Cin + 4 * P // 64 + 2 * P * Cout),
    )(img, mask, w_taps)

    y, nmo = pl.pallas_call(
        functools.partial(_bn_relu_kernel, cout=Cout, P=P),
        out_shape=(jax.ShapeDtypeStruct((N, Cout, H, W), jnp.float32),
                   jax.ShapeDtypeStruct((N, Cout, H, W), jnp.float32)),
        grid=(N,),
        in_specs=[pl.BlockSpec((1, Cout, H, W), lambda i: (i, 0, 0, 0)),
                  pl.BlockSpec((1, 1, H, W), lambda i: (i, 0, 0, 0)),
                  pl.BlockSpec((N, Cout, 2), lambda i: (0, 0, 0)),
                  pl.BlockSpec((Cout, 2), lambda i: (0, 0))],
        out_specs=(pl.BlockSpec((1, Cout, H, W), lambda i: (i, 0, 0, 0)),
                   pl.BlockSpec((1, Cout, H, W), lambda i: (i, 0, 0, 0))),
        compiler_params=cparams,
    )(out_t, nmask_t, stats, gb)

    return y, nmo


# speedup vs baseline: 11.9752x; 1.0408x over previous
import functools

import jax
import jax.numpy as jnp
from jax import lax
from jax.experimental import pallas as pl
from jax.experimental.pallas import tpu as pltpu


def _conv_stats_kernel(img_ref, mask_ref, w_ref,
                       out_ref, nmask_ref, stats_ref, *, cout, H, W):
    L = H * W
    PAD = 2 * W
    x4 = img_ref[0]
    m2 = mask_ref[0, 0]
    xm = (x4 * m2[None]).astype(jnp.bfloat16)
    xm = xm.reshape(x4.shape[0], L)

    xp = jnp.pad(xm, ((0, 0), (PAD, PAD)))
    lane = lax.broadcasted_iota(jnp.int32, (1, L + 2 * PAD), 1)
    wcol = (lane - PAD) % W
    zb = jnp.zeros((), jnp.bfloat16)
    xpL = jnp.where(wcol != (W - 1), xp, zb)
    xpR = jnp.where(wcol != 0, xp, zb)

    srcs = (xpL, xp, xpR) * 3
    shifts = [PAD + (t // 3 - 1) * W + (t % 3 - 1) for t in range(9)]
    acc = jnp.dot(w_ref[4], xm,
                  preferred_element_type=jnp.float32)
    for t in (0, 1, 2, 3, 5, 6, 7, 8):
        acc = acc + jnp.dot(w_ref[t], srcs[t][:, shifts[t]:shifts[t] + L],
                            preferred_element_type=jnp.float32)

    m = m2.reshape(1, L)
    mp = jnp.pad(m, ((0, 0), (PAD, PAD)))
    mpL = jnp.where(wcol != (W - 1), mp, 0.0)
    mpR = jnp.where(wcol != 0, mp, 0.0)
    msrcs = (mpL, mp, mpR) * 3
    ws = msrcs[0][:, shifts[0]:shifts[0] + L]
    for t in range(1, 9):
        ws = ws + msrcs[t][:, shifts[t]:shifts[t] + L]

    hole = ws == 0.0
    inv = jnp.where(hole, 0.0, 9.0 / jnp.where(hole, 1.0, ws))
    out = acc * inv

    s1 = jnp.sum(out, axis=1, keepdims=True)
    s2 = jnp.sum(out * out, axis=1, keepdims=True)
    stats_ref[...] = jnp.concatenate([s1, s2], axis=1)[None]

    out_ref[...] = out.astype(jnp.bfloat16)[None]
    nm = jnp.where(hole, 0.0, 1.0)
    nmask_ref[...] = nm.reshape(H, W)[None, None]


def _bn_relu_kernel(out_ref, nm_ref, stats_ref, gb_ref, y_ref, nmo_ref,
                    *, cout, H, W, P):
    s = jnp.sum(stats_ref[...], axis=0)
    mean = s[:, 0:1] / P
    var = jnp.maximum(s[:, 1:2] / P - mean * mean, 0.0)
    inv_std = lax.rsqrt(var + 1e-5)
    sc = gb_ref[:, 0:1] * inv_std
    sh = gb_ref[:, 1:2] - mean * sc
    x = out_ref[0].reshape(cout, H, W).astype(jnp.float32)
    y_ref[...] = jnp.maximum(x * sc[:, :, None] + sh[:, :, None], 0.0)[None]
    nm = nm_ref[0]
    nmo_ref[...] = jnp.broadcast_to(nm, x.shape)[None]


def kernel(img, mask, weight, gamma, beta):
    N, Cin, H, W = img.shape
    Cout = weight.shape[0]
    L = H * W
    P = N * L

    w_taps = weight.reshape(Cout, Cin, 9).transpose(2, 0, 1).astype(jnp.bfloat16)
    gb = jnp.stack([gamma, beta], axis=1).astype(jnp.float32)

    cparams = pltpu.CompilerParams(
        dimension_semantics=("arbitrary",),
        vmem_limit_bytes=100 * 1024 * 1024)

    kern1 = functools.partial(_conv_stats_kernel, cout=Cout, H=H, W=W)
    out_t, nmask_t, stats = pl.pallas_call(
        kern1,
        out_shape=(jax.ShapeDtypeStruct((N, Cout, L), jnp.bfloat16),
                   jax.ShapeDtypeStruct((N, 1, H, W), jnp.float32),
                   jax.ShapeDtypeStruct((N, Cout, 2), jnp.float32)),
        grid=(N,),
        in_specs=[pl.BlockSpec((1, Cin, H, W), lambda i: (i, 0, 0, 0)),
                  pl.BlockSpec((1, 1, H, W), lambda i: (i, 0, 0, 0)),
                  pl.BlockSpec((9, Cout, Cin), lambda i: (0, 0, 0))],
        out_specs=(pl.BlockSpec((1, Cout, L), lambda i: (i, 0, 0)),
                   pl.BlockSpec((1, 1, H, W), lambda i: (i, 0, 0, 0)),
                   pl.BlockSpec((1, Cout, 2), lambda i: (i, 0, 0))),
        compiler_params=cparams,
        cost_estimate=pl.CostEstimate(
            flops=2 * P * Cin * 9 * Cout,
            transcendentals=P,
            bytes_accessed=4 * P * Cin + 4 * P // 64 + 2 * P * Cout),
    )(img, mask, w_taps)

    y, nmo = pl.pallas_call(
        functools.partial(_bn_relu_kernel, cout=Cout, H=H, W=W, P=P),
        out_shape=(jax.ShapeDtypeStruct((N, Cout, H, W), jnp.float32),
                   jax.ShapeDtypeStruct((N, Cout, H, W), jnp.float32)),
        grid=(N,),
        in_specs=[pl.BlockSpec((1, Cout, L), lambda i: (i, 0, 0)),
                  pl.BlockSpec((1, 1, H, W), lambda i: (i, 0, 0, 0)),
                  pl.BlockSpec((N, Cout, 2), lambda i: (0, 0, 0)),
                  pl.BlockSpec((Cout, 2), lambda i: (0, 0))],
        out_specs=(pl.BlockSpec((1, Cout, H, W), lambda i: (i, 0, 0, 0)),
                   pl.BlockSpec((1, Cout, H, W), lambda i: (i, 0, 0, 0))),
        compiler_params=cparams,
    )(out_t, nmask_t, stats, gb)

    return y, nmo
